# baseline (speedup 1.0000x reference)
import functools

import jax
import jax.numpy as jnp
from jax import lax
from jax.experimental import pallas as pl
from jax.experimental.pallas import tpu as pltpu
from jax.experimental.pallas import tpu_sc as plsc


def _tc_body(
    cap,
    num_tokens,
    x_ref,
    wg_ref,
    idx1_ref,
    idx2_ref,
    g1_ref,
    g2_ref,
    l1_ref,
    l2r_ref,
    cnt_ref,
    laux_ref,
    ut_ref,
    log_acc,
    c1_acc,
    c2_acc,
    me_acc,
):
    blk = x_ref.shape[0]
    hd = x_ref.shape[1]
    ne = wg_ref.shape[0]
    i = pl.program_id(0)
    j = pl.program_id(1)
    nb = pl.num_programs(0)
    nk = pl.num_programs(1)

    @pl.when((i == 0) & (j == 0))
    def _init():
        c1_acc[...] = jnp.zeros_like(c1_acc)
        c2_acc[...] = jnp.zeros_like(c2_acc)
        me_acc[...] = jnp.zeros_like(me_acc)
        r = lax.broadcasted_iota(jnp.int32, (blk, blk), 0)
        c = lax.broadcasted_iota(jnp.int32, (blk, blk), 1)
        ut_ref[...] = (r <= c).astype(jnp.float32)

    part = lax.dot_general(
        wg_ref[:, pl.ds(j * hd, hd)],
        x_ref[...],
        (((1,), (1,)), ((), ())),
        preferred_element_type=jnp.float32,
    )

    @pl.when(j == 0)
    def _acc0():
        log_acc[...] = part

    @pl.when(j > 0)
    def _accn():
        log_acc[...] = log_acc[...] + part

    @pl.when(j == nk - 1)
    def _route():
        logits = log_acc[...]
        iota = lax.broadcasted_iota(jnp.int32, (ne, blk), 0)
        m1 = jnp.max(logits, axis=0)
        idx1 = jnp.min(jnp.where(logits == m1[None, :], iota, ne), axis=0)
        oh1 = iota == idx1[None, :]
        logits2 = jnp.where(oh1, -jnp.inf, logits)
        m2 = jnp.max(logits2, axis=0)
        idx2 = jnp.min(jnp.where(logits2 == m2[None, :], iota, ne), axis=0)
        oh2 = iota == idx2[None, :]

        e = jnp.exp(logits - m1[None, :])
        denom = jnp.sum(e, axis=0)
        g1 = 1.0 / denom
        g2 = jnp.exp(m2 - m1) / denom

        mh1 = oh1.astype(jnp.float32)
        mh2 = oh2.astype(jnp.float32)
        ut = ut_ref[...]
        cs1 = jnp.dot(mh1, ut, preferred_element_type=jnp.float32).astype(
            jnp.int32
        )
        cs2 = jnp.dot(mh2, ut, preferred_element_type=jnp.float32).astype(
            jnp.int32
        )
        c1_prev = c1_acc[...]
        c2_prev = c2_acc[...]
        loc1_full = cs1 - 1 + c1_prev
        loc2_full = cs2 - 1 + c2_prev
        loc1 = jnp.sum(jnp.where(oh1, loc1_full, 0), axis=0)
        loc2r = jnp.sum(jnp.where(oh2, loc2_full, 0), axis=0)
        loc1_s = jnp.where(loc1 < cap, loc1, 0)

        me_new = me_acc[...] + jnp.sum(
            e * (1.0 / denom)[None, :], axis=1, keepdims=True
        )
        c1_new = c1_prev + cs1[:, blk - 1 : blk]
        c2_new = c2_prev + cs2[:, blk - 1 : blk]
        c1_acc[...] = c1_new
        c2_acc[...] = c2_new
        me_acc[...] = me_new

        idx1_ref[0, 0, :] = idx1
        idx2_ref[0, 0, :] = idx2
        g1_ref[0, 0, :] = g1
        g2_ref[0, 0, :] = g2
        l1_ref[0, 0, :] = loc1_s
        l2r_ref[0, 0, :] = idx2 * 16384 + loc2r

        @pl.when(i == nb - 1)
        def _fin():
            cnt_ref[...] = c1_new
            ce = jnp.minimum(c1_new, cap).astype(jnp.float32)
            laux_ref[0, 0] = jnp.sum(me_new * ce) * jnp.float32(
                ne / (num_tokens * num_tokens)
            )


def _tc_stage(x, wg, cap, blk, nk):
    nt, d = x.shape
    ne = wg.shape[0]
    nb = nt // blk
    hd = d // nk
    tok3 = lambda dt: jax.ShapeDtypeStruct((nb, 1, blk), dt)
    tok_spec = pl.BlockSpec((1, 1, blk), lambda i, j: (i, 0, 0))
    return pl.pallas_call(
        functools.partial(_tc_body, cap, nt),
        grid=(nb, nk),
        in_specs=[
            pl.BlockSpec((blk, hd), lambda i, j: (i, j)),
            pl.BlockSpec((ne, d), lambda i, j: (0, 0)),
        ],
        out_specs=[
            tok_spec,
            tok_spec,
            tok_spec,
            tok_spec,
            tok_spec,
            tok_spec,
            pl.BlockSpec((ne, 1), lambda i, j: (0, 0)),
            pl.BlockSpec((1, 1), lambda i, j: (0, 0), memory_space=pltpu.SMEM),
        ],
        out_shape=[
            tok3(jnp.int32),
            tok3(jnp.int32),
            tok3(jnp.float32),
            tok3(jnp.float32),
            tok3(jnp.int32),
            tok3(jnp.int32),
            jax.ShapeDtypeStruct((ne, 1), jnp.int32),
            jax.ShapeDtypeStruct((1, 1), jnp.float32),
        ],
        scratch_shapes=[
            pltpu.VMEM((blk, blk), jnp.float32),
            pltpu.VMEM((ne, blk), jnp.float32),
            pltpu.VMEM((ne, 1), jnp.int32),
            pltpu.VMEM((ne, 1), jnp.int32),
            pltpu.VMEM((ne, 1), jnp.float32),
        ],
        compiler_params=pltpu.CompilerParams(
            dimension_semantics=("arbitrary", "arbitrary"),
        ),
    )(x, wg)


def _sc_fixup(packed, counts, cap):
    nt = packed.shape[0]
    ne = counts.shape[0]
    info = plsc.get_sparse_core_info()
    nc, ns, lanes = info.num_cores, info.num_subcores, info.num_lanes
    nw = nc * ns
    chunk = nt // nw
    mesh = plsc.VectorSubcoreMesh(core_axis_name="c", subcore_axis_name="s")

    @functools.partial(
        pl.kernel,
        mesh=mesh,
        out_type=jax.ShapeDtypeStruct((nt,), jnp.int32),
        scratch_types=[
            pltpu.VMEM((chunk,), jnp.int32),
            pltpu.VMEM((ne,), jnp.int32),
            pltpu.VMEM((chunk,), jnp.int32),
            pltpu.SemaphoreType.DMA,
            pltpu.SemaphoreType.DMA,
        ],
        compiler_params=pltpu.CompilerParams(needs_layout_passes=False),
    )
    def k(pk_hbm, cnt_hbm, out_hbm, pk_v, cnt_v, out_v, sem1, sem2):
        wid = lax.axis_index("s") * nc + lax.axis_index("c")
        base = wid * chunk
        cp1 = pltpu.async_copy(pk_hbm.at[pl.ds(base, chunk)], pk_v, sem1)
        cp2 = pltpu.async_copy(cnt_hbm, cnt_v, sem2)
        cp1.wait()
        cp2.wait()
        for j in range(chunk // lanes):
            sl = pl.ds(j * lanes, lanes)
            pk = pk_v[sl]
            idx = lax.shift_right_logical(pk, 14)
            raw = pk & 16383
            cv = plsc.load_gather(cnt_v, [idx])
            loc2 = raw + cv
            out_v[sl] = jnp.where(loc2 < cap, loc2, 0)
        pltpu.sync_copy(out_v, out_hbm.at[pl.ds(base, chunk)])

    return k(packed, counts)


def kernel(input, wg):
    nt, d = input.shape
    ne = wg.shape[0]
    cap = 2 * ((nt + ne - 1) // ne)
    blk = 1024
    nk = 2

    (idx1, idx2, g1, g2, l1, packed, counts, laux) = _tc_stage(
        input, wg, cap, blk, nk
    )
    idx1 = idx1.reshape(nt)
    idx2 = idx2.reshape(nt)
    g1 = g1.reshape(nt)
    g2 = g2.reshape(nt)
    l1 = l1.reshape(nt)
    packed = packed.reshape(nt)
    counts = counts.reshape(ne)

    l2 = _sc_fixup(packed, counts, cap)

    return (
        laux[0, 0],
        cap,
        ne,
        (idx1, idx2),
        (l1, l2),
        (g1, g2),
    )


# speedup vs baseline: 1.1675x; 1.1675x over previous
import functools

import jax
import jax.numpy as jnp
from jax import lax
from jax.experimental import pallas as pl
from jax.experimental.pallas import tpu as pltpu
from jax.experimental.pallas import tpu_sc as plsc


def _tc_body(
    cap,
    num_tokens,
    x_ref,
    wg_ref,
    idx1_ref,
    idx2_ref,
    g1_ref,
    g2_ref,
    l1_ref,
    l2r_ref,
    cnt_ref,
    laux_ref,
    ut_ref,
    log_acc,
    c1_acc,
    c2_acc,
    me_acc,
):
    blk = x_ref.shape[0]
    hd = x_ref.shape[1]
    ne = wg_ref.shape[0]
    i = pl.program_id(0)
    j = pl.program_id(1)
    nb = pl.num_programs(0)
    nk = pl.num_programs(1)

    @pl.when((i == 0) & (j == 0))
    def _init():
        c1_acc[...] = jnp.zeros_like(c1_acc)
        c2_acc[...] = jnp.zeros_like(c2_acc)
        me_acc[...] = jnp.zeros_like(me_acc)
        r = lax.broadcasted_iota(jnp.int32, (blk, blk), 0)
        c = lax.broadcasted_iota(jnp.int32, (blk, blk), 1)
        ut_ref[...] = (r <= c).astype(jnp.float32)

    part = lax.dot_general(
        wg_ref[:, pl.ds(j * hd, hd)],
        x_ref[...],
        (((1,), (1,)), ((), ())),
        preferred_element_type=jnp.float32,
    )

    @pl.when(j == 0)
    def _acc0():
        log_acc[...] = part

    @pl.when(j > 0)
    def _accn():
        log_acc[...] = log_acc[...] + part

    @pl.when(j == nk - 1)
    def _route():
        logits = log_acc[...]
        iota = lax.broadcasted_iota(jnp.int32, (ne, blk), 0)
        m1 = jnp.max(logits, axis=0)
        idx1 = jnp.min(jnp.where(logits == m1[None, :], iota, ne), axis=0)
        oh1 = iota == idx1[None, :]
        logits2 = jnp.where(oh1, -jnp.inf, logits)
        m2 = jnp.max(logits2, axis=0)
        idx2 = jnp.min(jnp.where(logits2 == m2[None, :], iota, ne), axis=0)
        oh2 = iota == idx2[None, :]

        e = jnp.exp(logits - m1[None, :])
        denom = jnp.sum(e, axis=0)
        g1 = 1.0 / denom
        g2 = jnp.exp(m2 - m1) / denom

        mh1 = oh1.astype(jnp.float32)
        mh2 = oh2.astype(jnp.float32)
        ut = ut_ref[...]
        cs1 = jnp.dot(mh1, ut, preferred_element_type=jnp.float32).astype(
            jnp.int32
        )
        cs2 = jnp.dot(mh2, ut, preferred_element_type=jnp.float32).astype(
            jnp.int32
        )
        c1_prev = c1_acc[...]
        c2_prev = c2_acc[...]
        loc1_full = cs1 - 1 + c1_prev
        loc2_full = cs2 - 1 + c2_prev
        loc1 = jnp.sum(jnp.where(oh1, loc1_full, 0), axis=0)
        loc2r = jnp.sum(jnp.where(oh2, loc2_full, 0), axis=0)
        loc1_s = jnp.where(loc1 < cap, loc1, 0)

        me_new = me_acc[...] + jnp.sum(
            e * (1.0 / denom)[None, :], axis=1, keepdims=True
        )
        c1_new = c1_prev + cs1[:, blk - 1 : blk]
        c2_new = c2_prev + cs2[:, blk - 1 : blk]
        c1_acc[...] = c1_new
        c2_acc[...] = c2_new
        me_acc[...] = me_new

        idx1_ref[0, 0, :] = idx1
        idx2_ref[0, 0, :] = idx2
        g1_ref[0, 0, :] = g1
        g2_ref[0, 0, :] = g2
        l1_ref[0, 0, :] = loc1_s
        l2r_ref[0, 0, :] = idx2 * 16384 + loc2r

        @pl.when(i == nb - 1)
        def _fin():
            cnt_ref[...] = c1_new
            ce = jnp.minimum(c1_new, cap).astype(jnp.float32)
            laux_ref[0, 0] = jnp.sum(me_new * ce) * jnp.float32(
                ne / (num_tokens * num_tokens)
            )


def _tc_stage(x, wg, cap, blk, nk):
    nt, d = x.shape
    ne = wg.shape[0]
    nb = nt // blk
    hd = d // nk
    tok3 = lambda dt: jax.ShapeDtypeStruct((nb, 1, blk), dt)
    tok_spec = pl.BlockSpec((1, 1, blk), lambda i, j: (i, 0, 0))
    return pl.pallas_call(
        functools.partial(_tc_body, cap, nt),
        grid=(nb, nk),
        in_specs=[
            pl.BlockSpec((blk, hd), lambda i, j: (i, j)),
            pl.BlockSpec((ne, d), lambda i, j: (0, 0)),
        ],
        out_specs=[
            tok_spec,
            tok_spec,
            tok_spec,
            tok_spec,
            tok_spec,
            tok_spec,
            pl.BlockSpec((ne, 1), lambda i, j: (0, 0)),
            pl.BlockSpec((1, 1), lambda i, j: (0, 0), memory_space=pltpu.SMEM),
        ],
        out_shape=[
            tok3(jnp.int32),
            tok3(jnp.int32),
            tok3(jnp.float32),
            tok3(jnp.float32),
            tok3(jnp.int32),
            tok3(jnp.int32),
            jax.ShapeDtypeStruct((ne, 1), jnp.int32),
            jax.ShapeDtypeStruct((1, 1), jnp.float32),
        ],
        scratch_shapes=[
            pltpu.VMEM((blk, blk), jnp.float32),
            pltpu.VMEM((ne, blk), jnp.float32),
            pltpu.VMEM((ne, 1), jnp.int32),
            pltpu.VMEM((ne, 1), jnp.int32),
            pltpu.VMEM((ne, 1), jnp.float32),
        ],
        compiler_params=pltpu.CompilerParams(
            dimension_semantics=("arbitrary", "arbitrary"),
        ),
    )(x, wg)


def _sc_fixup(packed, counts, cap):
    nt = packed.shape[0]
    ne = counts.shape[0]
    info = plsc.get_sparse_core_info()
    nc, ns, lanes = info.num_cores, info.num_subcores, info.num_lanes
    nw = nc * ns
    chunk = nt // nw
    mesh = plsc.VectorSubcoreMesh(core_axis_name="c", subcore_axis_name="s")

    @functools.partial(
        pl.kernel,
        mesh=mesh,
        out_type=jax.ShapeDtypeStruct((nt,), jnp.int32),
        scratch_types=[
            pltpu.VMEM((chunk,), jnp.int32),
            pltpu.VMEM((ne,), jnp.int32),
            pltpu.VMEM((chunk,), jnp.int32),
            pltpu.SemaphoreType.DMA,
            pltpu.SemaphoreType.DMA,
        ],
        compiler_params=pltpu.CompilerParams(needs_layout_passes=False),
    )
    def k(pk_hbm, cnt_hbm, out_hbm, pk_v, cnt_v, out_v, sem1, sem2):
        wid = lax.axis_index("s") * nc + lax.axis_index("c")
        base = wid * chunk
        cp1 = pltpu.async_copy(pk_hbm.at[pl.ds(base, chunk)], pk_v, sem1)
        cp2 = pltpu.async_copy(cnt_hbm, cnt_v, sem2)
        cp1.wait()
        cp2.wait()
        for j in range(chunk // lanes):
            sl = pl.ds(j * lanes, lanes)
            pk = pk_v[sl]
            idx = lax.shift_right_logical(pk, 14)
            raw = pk & 16383
            cv = plsc.load_gather(cnt_v, [idx])
            loc2 = raw + cv
            out_v[sl] = jnp.where(loc2 < cap, loc2, 0)
        pltpu.sync_copy(out_v, out_hbm.at[pl.ds(base, chunk)])

    return k(packed, counts)


def kernel(input, wg):
    nt, d = input.shape
    ne = wg.shape[0]
    cap = 2 * ((nt + ne - 1) // ne)
    blk = 1024
    nk = 1

    (idx1, idx2, g1, g2, l1, packed, counts, laux) = _tc_stage(
        input, wg, cap, blk, nk
    )
    idx1 = idx1.reshape(nt)
    idx2 = idx2.reshape(nt)
    g1 = g1.reshape(nt)
    g2 = g2.reshape(nt)
    l1 = l1.reshape(nt)
    packed = packed.reshape(nt)
    counts = counts.reshape(ne)

    l2 = _sc_fixup(packed, counts, cap)

    return (
        laux[0, 0],
        cap,
        ne,
        (idx1, idx2),
        (l1, l2),
        (g1, g2),
    )
